# baseline (speedup 1.0000x reference)
import functools

import jax
import jax.numpy as jnp
from jax import lax
from jax.experimental import pallas as pl
from jax.experimental.pallas import tpu as pltpu
from jax.experimental.pallas import tpu_sc as plsc

N = 10000
D = 128
E = 160000
NC, NS, L = 2, 16, 16
NBUF = 3
CHUNK = 104
CPT = 99
EPT = NS * CPT * CHUNK
RACC = N + 8
RPT = 632
LASTR = N - (NS - 1) * RPT
MROWS = 400


def _sc_agg(xcat, src_idx, dst_idx, x_op, x_mach):
    mesh = plsc.VectorSubcoreMesh(core_axis_name="c", subcore_axis_name="s")

    @functools.partial(
        pl.kernel,
        mesh=mesh,
        out_type=jax.ShapeDtypeStruct((NC * N, D), jnp.float32),
        scratch_types=(
            [pltpu.VMEM((CPT * CHUNK,), jnp.int32),
             pltpu.VMEM((NBUF, CHUNK), jnp.int32)]
            + [pltpu.VMEM((CHUNK, D), jnp.float32)] * NBUF
            + [pltpu.VMEM_SHARED((RACC, D), jnp.float32)]
            + [pltpu.SemaphoreType.DMA] * (2 * NBUF + 2)
        ),
    )
    def k(xcat_hbm, src_hbm, dst_hbm, xop_hbm, xmach_hbm, out_hbm,
          src_v, dring, *rest):
        rows_l = rest[:NBUF]
        accum = rest[NBUF]
        sg_l = rest[NBUF + 1:NBUF + 1 + NBUF]
        sd_l = rest[NBUF + 1 + NBUF:NBUF + 1 + 2 * NBUF]
        s_stage, s_init = rest[NBUF + 1 + 2 * NBUF:]
        c = lax.axis_index("c")
        s = lax.axis_index("s")
        w = c * NS + s

        stage_cp = pltpu.async_copy(src_hbm.at[w], src_v, s_stage)

        for cc, xd in ((0, xmach_hbm), (1, xop_hbm)):
            @pl.when((c == cc) & (s < NS - 1))
            def _(xd=xd):
                pltpu.async_copy(xd.at[pl.ds(s * RPT, RPT)],
                                 accum.at[pl.ds(s * RPT, RPT)], s_init)

            @pl.when((c == cc) & (s == NS - 1))
            def _(xd=xd):
                pltpu.async_copy(xd.at[pl.ds((NS - 1) * RPT, LASTR)],
                                 accum.at[pl.ds((NS - 1) * RPT, LASTR)], s_init)

        def gidx(j):
            return src_v.at[pl.ds(j * CHUNK, CHUNK)]

        stage_cp.wait()
        bufs = tuple(zip(rows_l, sg_l, sd_l))
        for b, (rows, sg, sd) in enumerate(bufs):
            pltpu.async_copy(xcat_hbm.at[gidx(b)], rows, sg)
            pltpu.async_copy(dst_hbm.at[w * CPT + b], dring.at[b], sd)

        @pl.when(s < NS - 1)
        def _():
            pltpu.make_async_copy(
                xmach_hbm.at[pl.ds(s * RPT, RPT)],
                accum.at[pl.ds(s * RPT, RPT)], s_init).wait()

        @pl.when(s == NS - 1)
        def _():
            pltpu.make_async_copy(
                xmach_hbm.at[pl.ds((NS - 1) * RPT, LASTR)],
                accum.at[pl.ds((NS - 1) * RPT, LASTR)], s_init).wait()

        plsc.subcore_barrier()

        def body(g, carry):
            j = NBUF * g
            for b, (rows, sg, sd) in enumerate(bufs):
                pltpu.make_async_copy(xcat_hbm.at[gidx(j + b)], rows, sg).wait()
                pltpu.make_async_copy(dst_hbm.at[w * CPT + j + b],
                                      dring.at[b], sd).wait()
                pltpu.sync_copy(rows, accum.at[dring.at[b]], add=True)

                @pl.when(j + b + NBUF < CPT)
                def _():
                    pltpu.async_copy(xcat_hbm.at[gidx(j + b + NBUF)], rows, sg)
                    pltpu.async_copy(dst_hbm.at[w * CPT + j + b + NBUF],
                                     dring.at[b], sd)

            return carry

        lax.fori_loop(0, CPT // NBUF, body, 0)
        plsc.subcore_barrier()

        @pl.when(s < NS - 1)
        def _():
            pltpu.sync_copy(accum.at[pl.ds(s * RPT, RPT)],
                            out_hbm.at[pl.ds(c * N + s * RPT, RPT)])

        @pl.when(s == NS - 1)
        def _():
            pltpu.sync_copy(accum.at[pl.ds((NS - 1) * RPT, LASTR)],
                            out_hbm.at[pl.ds(c * N + (NS - 1) * RPT, LASTR)])

    return k(xcat, src_idx, dst_idx, x_op, x_mach)


def _tc_mlp_body(agg0_ref, agg1_ref, xm_ref, xo_ref,
                 w1_ref, b1_ref, w2_ref, b2_ref, eps_ref,
                 o0_ref, o1_ref):
    def mlp(xin, t):
        h = jnp.dot(xin, w1_ref[t], preferred_element_type=jnp.float32)
        h = jnp.maximum(h + b1_ref[t], 0.0)
        y = jnp.dot(h, w2_ref[t], preferred_element_type=jnp.float32)
        return jnp.maximum(y + b2_ref[t], 0.0)

    o0_ref[...] = mlp(agg0_ref[...] + eps_ref[0] * xm_ref[...], 0)
    o1_ref[...] = mlp(agg1_ref[...] + eps_ref[1] * xo_ref[...], 1)


def _tc_mlp(agg, x_mach, x_op, w1s, b1s, w2s, b2s, epss):
    nb = N // MROWS
    out = pl.pallas_call(
        _tc_mlp_body,
        grid=(nb,),
        in_specs=[
            pl.BlockSpec((MROWS, D), lambda i: (i, 0)),
            pl.BlockSpec((MROWS, D), lambda i, _nb=nb: (i + _nb, 0)),
            pl.BlockSpec((MROWS, D), lambda i: (i, 0)),
            pl.BlockSpec((MROWS, D), lambda i: (i, 0)),
            pl.BlockSpec((NC, D, D), lambda i: (0, 0, 0)),
            pl.BlockSpec((NC, 1, D), lambda i: (0, 0, 0)),
            pl.BlockSpec((NC, D, D), lambda i: (0, 0, 0)),
            pl.BlockSpec((NC, 1, D), lambda i: (0, 0, 0)),
            pl.BlockSpec(memory_space=pltpu.SMEM),
        ],
        out_specs=[
            pl.BlockSpec((MROWS, D), lambda i: (i, 0)),
            pl.BlockSpec((MROWS, D), lambda i: (i, 0)),
        ],
        out_shape=[
            jax.ShapeDtypeStruct((N, D), jnp.float32),
            jax.ShapeDtypeStruct((N, D), jnp.float32),
        ],
    )(agg, agg, x_mach, x_op, w1s, b1s, w2s, b2s, epss)
    return out


def _fold_bn(W1, b1, g1, be1, rm1, rv1, W2, b2, g2, be2, rm2, rv2):
    s1 = g1 * lax.rsqrt(rv1 + 1e-5)
    s2 = g2 * lax.rsqrt(rv2 + 1e-5)
    return (W1 * s1[None, :], (b1 - rm1) * s1 + be1,
            W2 * s2[None, :], (b2 - rm2) * s2 + be2)


def kernel(x_op, x_mach, ei_om, ei_mo,
           W1_op, b1_op, g1_op, be1_op, rm1_op, rv1_op,
           W2_op, b2_op, g2_op, be2_op, rm2_op, rv2_op,
           W1_mach, b1_mach, g1_mach, be1_mach, rm1_mach, rv1_mach,
           W2_mach, b2_mach, g2_mach, be2_mach, rm2_mach, rv2_mach,
           eps_om, eps_mo):
    pad = EPT - E
    zpad_i = jnp.zeros((pad,), jnp.int32)
    dpad_i = jnp.full((pad,), N, jnp.int32)

    xcat = jnp.concatenate([x_op, x_mach], axis=0)
    src_all = jnp.concatenate(
        [ei_om[0], zpad_i, ei_mo[0] + N, zpad_i]).reshape(NC * NS, CPT * CHUNK)
    dst_all = jnp.concatenate(
        [ei_om[1], dpad_i, ei_mo[1], dpad_i]).reshape(NC * NS * CPT, CHUNK)

    agg = _sc_agg(xcat, src_all, dst_all, x_op, x_mach)

    w1f_op, b1f_op, w2f_op, b2f_op = _fold_bn(
        W1_op, b1_op, g1_op, be1_op, rm1_op, rv1_op,
        W2_op, b2_op, g2_op, be2_op, rm2_op, rv2_op)
    w1f_m, b1f_m, w2f_m, b2f_m = _fold_bn(
        W1_mach, b1_mach, g1_mach, be1_mach, rm1_mach, rv1_mach,
        W2_mach, b2_mach, g2_mach, be2_mach, rm2_mach, rv2_mach)

    w1s = jnp.stack([w1f_op, w1f_m])
    b1s = jnp.stack([b1f_op, b1f_m])[:, None, :]
    w2s = jnp.stack([w2f_op, w2f_m])
    b2s = jnp.stack([b2f_op, b2f_m])[:, None, :]
    epss = jnp.stack([eps_om, eps_mo])

    out_mach, out_op = _tc_mlp(agg, x_mach, x_op, w1s, b1s, w2s, b2s, epss)
    return (out_op, out_mach)


# speedup vs baseline: 2.0108x; 2.0108x over previous
import functools

import jax
import jax.numpy as jnp
from jax import lax
from jax.experimental import pallas as pl
from jax.experimental.pallas import tpu as pltpu
from jax.experimental.pallas import tpu_sc as plsc

N = 10000
D = 128
E = 160000
NC, NS, L = 2, 16, 16
NBUF = 3
CHUNK = 96
CPT = 105
EPT = NS * CPT * CHUNK
RACC = N + 8
RPT = 632
LASTR = N - (NS - 1) * RPT
MROWS = 400


def _sc_agg(xcat, src_idx, dst_idx, x_op, x_mach):
    mesh = plsc.VectorSubcoreMesh(core_axis_name="c", subcore_axis_name="s")

    @functools.partial(
        pl.kernel,
        mesh=mesh,
        out_type=jax.ShapeDtypeStruct((NC * N, D), jnp.float32),
        scratch_types=(
            [pltpu.VMEM((CPT * CHUNK,), jnp.int32),
             pltpu.VMEM((NBUF, CHUNK), jnp.int32)]
            + [pltpu.VMEM((CHUNK, D), jnp.float32)] * NBUF
            + [pltpu.VMEM_SHARED((RACC, D), jnp.float32)]
            + [pltpu.SemaphoreType.DMA] * (2 * NBUF + 2)
        ),
    )
    def k(xcat_hbm, src_hbm, dst_hbm, xop_hbm, xmach_hbm, out_hbm,
          src_v, dring, *rest):
        rows_l = rest[:NBUF]
        accum = rest[NBUF]
        sg_l = rest[NBUF + 1:NBUF + 1 + NBUF]
        sd_l = rest[NBUF + 1 + NBUF:NBUF + 1 + 2 * NBUF]
        s_stage, s_init = rest[NBUF + 1 + 2 * NBUF:]
        c = lax.axis_index("c")
        s = lax.axis_index("s")
        w = c * NS + s

        stage_cp = pltpu.async_copy(src_hbm.at[w], src_v, s_stage)

        for cc, xd in ((0, xmach_hbm), (1, xop_hbm)):
            @pl.when((c == cc) & (s < NS - 1))
            def _(xd=xd):
                pltpu.async_copy(xd.at[pl.ds(s * RPT, RPT)],
                                 accum.at[pl.ds(s * RPT, RPT)], s_init)

            @pl.when((c == cc) & (s == NS - 1))
            def _(xd=xd):
                pltpu.async_copy(xd.at[pl.ds((NS - 1) * RPT, LASTR)],
                                 accum.at[pl.ds((NS - 1) * RPT, LASTR)], s_init)

        def gidx(j):
            return src_v.at[pl.ds(j * CHUNK, CHUNK)]

        stage_cp.wait()
        bufs = tuple(zip(rows_l, sg_l, sd_l))
        for b, (rows, sg, sd) in enumerate(bufs):
            pltpu.async_copy(xcat_hbm.at[gidx(b)], rows, sg)
            pltpu.async_copy(dst_hbm.at[w * CPT + b], dring.at[b], sd)

        @pl.when(s < NS - 1)
        def _():
            pltpu.make_async_copy(
                xmach_hbm.at[pl.ds(s * RPT, RPT)],
                accum.at[pl.ds(s * RPT, RPT)], s_init).wait()

        @pl.when(s == NS - 1)
        def _():
            pltpu.make_async_copy(
                xmach_hbm.at[pl.ds((NS - 1) * RPT, LASTR)],
                accum.at[pl.ds((NS - 1) * RPT, LASTR)], s_init).wait()

        plsc.subcore_barrier()

        def body(g, carry):
            j = NBUF * g
            for b, (rows, sg, sd) in enumerate(bufs):
                pltpu.make_async_copy(xcat_hbm.at[gidx(j + b)], rows, sg).wait()
                pltpu.make_async_copy(dst_hbm.at[w * CPT + j + b],
                                      dring.at[b], sd).wait()
                pltpu.sync_copy(rows, accum.at[dring.at[b]], add=True)

                @pl.when(j + b + NBUF < CPT)
                def _():
                    pltpu.async_copy(xcat_hbm.at[gidx(j + b + NBUF)], rows, sg)
                    pltpu.async_copy(dst_hbm.at[w * CPT + j + b + NBUF],
                                     dring.at[b], sd)

            return carry

        lax.fori_loop(0, CPT // NBUF, body, 0)
        plsc.subcore_barrier()

        @pl.when(s < NS - 1)
        def _():
            pltpu.sync_copy(accum.at[pl.ds(s * RPT, RPT)],
                            out_hbm.at[pl.ds(c * N + s * RPT, RPT)])

        @pl.when(s == NS - 1)
        def _():
            pltpu.sync_copy(accum.at[pl.ds((NS - 1) * RPT, LASTR)],
                            out_hbm.at[pl.ds(c * N + (NS - 1) * RPT, LASTR)])

    return k(xcat, src_idx, dst_idx, x_op, x_mach)


def _tc_mlp_body(agg0_ref, agg1_ref, xm_ref, xo_ref,
                 w1_ref, b1_ref, w2_ref, b2_ref, eps_ref,
                 o0_ref, o1_ref):
    def mlp(xin, t):
        h = jnp.dot(xin, w1_ref[t], preferred_element_type=jnp.float32)
        h = jnp.maximum(h + b1_ref[t], 0.0)
        y = jnp.dot(h, w2_ref[t], preferred_element_type=jnp.float32)
        return jnp.maximum(y + b2_ref[t], 0.0)

    o0_ref[...] = mlp(agg0_ref[...] + eps_ref[0] * xm_ref[...], 0)
    o1_ref[...] = mlp(agg1_ref[...] + eps_ref[1] * xo_ref[...], 1)


def _tc_mlp(agg, x_mach, x_op, w1s, b1s, w2s, b2s, epss):
    nb = N // MROWS
    out = pl.pallas_call(
        _tc_mlp_body,
        grid=(nb,),
        in_specs=[
            pl.BlockSpec((MROWS, D), lambda i: (i, 0)),
            pl.BlockSpec((MROWS, D), lambda i, _nb=nb: (i + _nb, 0)),
            pl.BlockSpec((MROWS, D), lambda i: (i, 0)),
            pl.BlockSpec((MROWS, D), lambda i: (i, 0)),
            pl.BlockSpec((NC, D, D), lambda i: (0, 0, 0)),
            pl.BlockSpec((NC, 1, D), lambda i: (0, 0, 0)),
            pl.BlockSpec((NC, D, D), lambda i: (0, 0, 0)),
            pl.BlockSpec((NC, 1, D), lambda i: (0, 0, 0)),
            pl.BlockSpec(memory_space=pltpu.SMEM),
        ],
        out_specs=[
            pl.BlockSpec((MROWS, D), lambda i: (i, 0)),
            pl.BlockSpec((MROWS, D), lambda i: (i, 0)),
        ],
        out_shape=[
            jax.ShapeDtypeStruct((N, D), jnp.float32),
            jax.ShapeDtypeStruct((N, D), jnp.float32),
        ],
    )(agg, agg, x_mach, x_op, w1s, b1s, w2s, b2s, epss)
    return out


def _fold_bn(W1, b1, g1, be1, rm1, rv1, W2, b2, g2, be2, rm2, rv2):
    s1 = g1 * lax.rsqrt(rv1 + 1e-5)
    s2 = g2 * lax.rsqrt(rv2 + 1e-5)
    return (W1 * s1[None, :], (b1 - rm1) * s1 + be1,
            W2 * s2[None, :], (b2 - rm2) * s2 + be2)


def kernel(x_op, x_mach, ei_om, ei_mo,
           W1_op, b1_op, g1_op, be1_op, rm1_op, rv1_op,
           W2_op, b2_op, g2_op, be2_op, rm2_op, rv2_op,
           W1_mach, b1_mach, g1_mach, be1_mach, rm1_mach, rv1_mach,
           W2_mach, b2_mach, g2_mach, be2_mach, rm2_mach, rv2_mach,
           eps_om, eps_mo):
    pad = EPT - E
    zpad_i = jnp.zeros((pad,), jnp.int32)
    dpad_i = jnp.full((pad,), N, jnp.int32)

    xcat = jnp.concatenate([x_op, x_mach], axis=0)
    src_all = jnp.concatenate(
        [ei_om[0], zpad_i, ei_mo[0] + N, zpad_i]).reshape(NC * NS, CPT * CHUNK)
    dst_all = jnp.concatenate(
        [ei_om[1], dpad_i, ei_mo[1], dpad_i]).reshape(NC * NS * CPT, CHUNK)

    agg = _sc_agg(xcat, src_all, dst_all, x_op, x_mach)

    w1f_op, b1f_op, w2f_op, b2f_op = _fold_bn(
        W1_op, b1_op, g1_op, be1_op, rm1_op, rv1_op,
        W2_op, b2_op, g2_op, be2_op, rm2_op, rv2_op)
    w1f_m, b1f_m, w2f_m, b2f_m = _fold_bn(
        W1_mach, b1_mach, g1_mach, be1_mach, rm1_mach, rv1_mach,
        W2_mach, b2_mach, g2_mach, be2_mach, rm2_mach, rv2_mach)

    w1s = jnp.stack([w1f_op, w1f_m])
    b1s = jnp.stack([b1f_op, b1f_m])[:, None, :]
    w2s = jnp.stack([w2f_op, w2f_m])
    b2s = jnp.stack([b2f_op, b2f_m])[:, None, :]
    epss = jnp.stack([eps_om, eps_mo])

    out_mach, out_op = _tc_mlp(agg, x_mach, x_op, w1s, b1s, w2s, b2s, epss)
    return (out_op, out_mach)


# speedup vs baseline: 3.6258x; 1.8031x over previous
import functools

import jax
import jax.numpy as jnp
from jax import lax
from jax.experimental import pallas as pl
from jax.experimental.pallas import tpu as pltpu
from jax.experimental.pallas import tpu_sc as plsc

N = 10000
D = 128
E = 160000
NC, NS, L = 2, 16, 16
NBUF = 3
CHUNK = 96
PTE = E // NS
CPT = PTE // CHUNK
TAIL = PTE - CPT * CHUNK
RPT = 632
LASTR = N - (NS - 1) * RPT
MROWS = 400


def _sc_agg(x_op, x_mach, s_om, d_om, s_mo, d_mo):
    mesh = plsc.VectorSubcoreMesh(core_axis_name="c", subcore_axis_name="s")

    @functools.partial(
        pl.kernel,
        mesh=mesh,
        out_type=jax.ShapeDtypeStruct((NC * N, D), jnp.float32),
        scratch_types=(
            [pltpu.VMEM((PTE,), jnp.int32),
             pltpu.VMEM((NBUF, CHUNK), jnp.int32),
             pltpu.VMEM((1, TAIL), jnp.int32)]
            + [pltpu.VMEM((CHUNK, D), jnp.float32)] * NBUF
            + [pltpu.VMEM_SHARED((N, D), jnp.float32)]
            + [pltpu.SemaphoreType.DMA] * (2 * NBUF + 2)
        ),
    )
    def k(xop_hbm, xmach_hbm, som_hbm, dom_hbm, smo_hbm, dmo_hbm, out_hbm,
          src_v, dring, dtail, *rest):
        rows_l = rest[:NBUF]
        accum = rest[NBUF]
        sg_l = rest[NBUF + 1:NBUF + 1 + NBUF]
        sd_l = rest[NBUF + 1 + NBUF:NBUF + 1 + 2 * NBUF]
        s_stage, s_init = rest[NBUF + 1 + 2 * NBUF:]
        c = lax.axis_index("c")
        s = lax.axis_index("s")
        bufs = tuple(zip(rows_l, sg_l, sd_l))

        def pipeline(table, src_hbm, dst_hbm, xd_hbm):
            ebase = s * PTE

            stage_cp = pltpu.async_copy(
                src_hbm.at[pl.ds(ebase, PTE)], src_v, s_stage)

            @pl.when(s < NS - 1)
            def _():
                pltpu.async_copy(xd_hbm.at[pl.ds(s * RPT, RPT)],
                                 accum.at[pl.ds(s * RPT, RPT)], s_init)

            @pl.when(s == NS - 1)
            def _():
                pltpu.async_copy(xd_hbm.at[pl.ds((NS - 1) * RPT, LASTR)],
                                 accum.at[pl.ds((NS - 1) * RPT, LASTR)], s_init)

            def gidx(j):
                return src_v.at[pl.ds(j * CHUNK, CHUNK)]

            stage_cp.wait()
            for b, (rows, sg, sd) in enumerate(bufs):
                pltpu.async_copy(table.at[gidx(b)], rows, sg)
                pltpu.async_copy(dst_hbm.at[pl.ds(ebase + b * CHUNK, CHUNK)],
                                 dring.at[b], sd)

            @pl.when(s < NS - 1)
            def _():
                pltpu.make_async_copy(
                    xd_hbm.at[pl.ds(s * RPT, RPT)],
                    accum.at[pl.ds(s * RPT, RPT)], s_init).wait()

            @pl.when(s == NS - 1)
            def _():
                pltpu.make_async_copy(
                    xd_hbm.at[pl.ds((NS - 1) * RPT, LASTR)],
                    accum.at[pl.ds((NS - 1) * RPT, LASTR)], s_init).wait()

            plsc.subcore_barrier()

            def step(j, b):
                rows, sg, sd = bufs[b]
                pltpu.make_async_copy(table.at[gidx(j)], rows, sg).wait()
                pltpu.make_async_copy(
                    dst_hbm.at[pl.ds(ebase + j * CHUNK, CHUNK)],
                    dring.at[b], sd).wait()
                pltpu.sync_copy(rows, accum.at[dring.at[b]], add=True)

                @pl.when(j + NBUF < CPT)
                def _():
                    pltpu.async_copy(table.at[gidx(j + NBUF)], rows, sg)
                    pltpu.async_copy(
                        dst_hbm.at[pl.ds(ebase + (j + NBUF) * CHUNK, CHUNK)],
                        dring.at[b], sd)

            def body(g, carry):
                j = NBUF * g
                for b in range(NBUF):
                    step(j + b, b)
                return carry

            lax.fori_loop(0, CPT // NBUF, body, 0)
            step(CPT // NBUF * NBUF, 0)
            step(CPT // NBUF * NBUF + 1, 1)

            rows, sg, sd = bufs[2]
            tcp = pltpu.async_copy(
                table.at[src_v.at[pl.ds(CPT * CHUNK, TAIL)]],
                rows.at[pl.ds(0, TAIL)], sg)
            pltpu.sync_copy(dst_hbm.at[pl.ds(ebase + CPT * CHUNK, TAIL)],
                            dtail.at[0])
            tcp.wait()
            pltpu.sync_copy(rows.at[pl.ds(0, TAIL)],
                            accum.at[dtail.at[0]], add=True)

            plsc.subcore_barrier()

            @pl.when(s < NS - 1)
            def _():
                pltpu.sync_copy(accum.at[pl.ds(s * RPT, RPT)],
                                out_hbm.at[pl.ds(c * N + s * RPT, RPT)])

            @pl.when(s == NS - 1)
            def _():
                pltpu.sync_copy(
                    accum.at[pl.ds((NS - 1) * RPT, LASTR)],
                    out_hbm.at[pl.ds(c * N + (NS - 1) * RPT, LASTR)])

        @pl.when(c == 0)
        def _():
            pipeline(xop_hbm, som_hbm, dom_hbm, xmach_hbm)

        @pl.when(c == 1)
        def _():
            pipeline(xmach_hbm, smo_hbm, dmo_hbm, xop_hbm)

    return k(x_op, x_mach, s_om, d_om, s_mo, d_mo)


def _tc_mlp_body(agg0_ref, agg1_ref, xm_ref, xo_ref,
                 w1_ref, b1_ref, w2_ref, b2_ref, eps_ref,
                 o0_ref, o1_ref):
    def mlp(xin, t):
        h = jnp.dot(xin, w1_ref[t], preferred_element_type=jnp.float32)
        h = jnp.maximum(h + b1_ref[t], 0.0)
        y = jnp.dot(h, w2_ref[t], preferred_element_type=jnp.float32)
        return jnp.maximum(y + b2_ref[t], 0.0)

    o0_ref[...] = mlp(agg0_ref[...] + eps_ref[0] * xm_ref[...], 0)
    o1_ref[...] = mlp(agg1_ref[...] + eps_ref[1] * xo_ref[...], 1)


def _tc_mlp(agg, x_mach, x_op, w1s, b1s, w2s, b2s, epss):
    nb = N // MROWS
    out = pl.pallas_call(
        _tc_mlp_body,
        grid=(nb,),
        in_specs=[
            pl.BlockSpec((MROWS, D), lambda i: (i, 0)),
            pl.BlockSpec((MROWS, D), lambda i, _nb=nb: (i + _nb, 0)),
            pl.BlockSpec((MROWS, D), lambda i: (i, 0)),
            pl.BlockSpec((MROWS, D), lambda i: (i, 0)),
            pl.BlockSpec((NC, D, D), lambda i: (0, 0, 0)),
            pl.BlockSpec((NC, 1, D), lambda i: (0, 0, 0)),
            pl.BlockSpec((NC, D, D), lambda i: (0, 0, 0)),
            pl.BlockSpec((NC, 1, D), lambda i: (0, 0, 0)),
            pl.BlockSpec(memory_space=pltpu.SMEM),
        ],
        out_specs=[
            pl.BlockSpec((MROWS, D), lambda i: (i, 0)),
            pl.BlockSpec((MROWS, D), lambda i: (i, 0)),
        ],
        out_shape=[
            jax.ShapeDtypeStruct((N, D), jnp.float32),
            jax.ShapeDtypeStruct((N, D), jnp.float32),
        ],
    )(agg, agg, x_mach, x_op, w1s, b1s, w2s, b2s, epss)
    return out


def _fold_bn(W1, b1, g1, be1, rm1, rv1, W2, b2, g2, be2, rm2, rv2):
    s1 = g1 * lax.rsqrt(rv1 + 1e-5)
    s2 = g2 * lax.rsqrt(rv2 + 1e-5)
    return (W1 * s1[None, :], (b1 - rm1) * s1 + be1,
            W2 * s2[None, :], (b2 - rm2) * s2 + be2)


def kernel(x_op, x_mach, ei_om, ei_mo,
           W1_op, b1_op, g1_op, be1_op, rm1_op, rv1_op,
           W2_op, b2_op, g2_op, be2_op, rm2_op, rv2_op,
           W1_mach, b1_mach, g1_mach, be1_mach, rm1_mach, rv1_mach,
           W2_mach, b2_mach, g2_mach, be2_mach, rm2_mach, rv2_mach,
           eps_om, eps_mo):
    agg = _sc_agg(x_op, x_mach, ei_om[0], ei_om[1], ei_mo[0], ei_mo[1])

    w1f_op, b1f_op, w2f_op, b2f_op = _fold_bn(
        W1_op, b1_op, g1_op, be1_op, rm1_op, rv1_op,
        W2_op, b2_op, g2_op, be2_op, rm2_op, rv2_op)
    w1f_m, b1f_m, w2f_m, b2f_m = _fold_bn(
        W1_mach, b1_mach, g1_mach, be1_mach, rm1_mach, rv1_mach,
        W2_mach, b2_mach, g2_mach, be2_mach, rm2_mach, rv2_mach)

    w1s = jnp.stack([w1f_op, w1f_m])
    b1s = jnp.stack([b1f_op, b1f_m])[:, None, :]
    w2s = jnp.stack([w2f_op, w2f_m])
    b2s = jnp.stack([b2f_op, b2f_m])[:, None, :]
    epss = jnp.stack([eps_om, eps_mo])

    out_mach, out_op = _tc_mlp(agg, x_mach, x_op, w1s, b1s, w2s, b2s, epss)
    return (out_op, out_mach)
